# baseline (speedup 1.0000x reference)
import jax
import jax.numpy as jnp
from jax.experimental import pallas as pl

B = 256
N = 512
OUT_LEN = N * (N + 1) // 2

_OFFR = [r * N - r * (r - 1) // 2 for r in range(N + 1)]

_TC_PANELS = [
    (0, 128, 0, 512),
    (128, 128, 128, 128),
    (128, 128, 256, 256),
    (256, 128, 256, 256),
    (384, 128, 384, 128),
]

_TC_MB = 8


def _tc_body(*refs):
    x_refs, out_ref = refs[:-1], refs[-1]
    for bb in range(_TC_MB):
        for r in range(N - 1, -1, -1):
            for x_ref, (r0, nr, c0, nc) in zip(x_refs, _TC_PANELS):
                if not (r0 <= r < r0 + nr) or c0 + nc <= r:
                    continue
                out_ref[bb, 0, pl.ds(_OFFR[r] - r + c0, nc)] = (
                    x_ref[bb, r - r0, :])


def kernel(x):
    def spec(r0, nr, c0, nc):
        return pl.BlockSpec(
            (_TC_MB, nr, nc), lambda b, r0=r0, nr=nr, c0=c0, nc=nc:
            (b, r0 // nr, c0 // nc))

    out = pl.pallas_call(
        _tc_body,
        grid=(B // _TC_MB,),
        in_specs=[spec(*p) for p in _TC_PANELS],
        out_specs=pl.BlockSpec((_TC_MB, 1, OUT_LEN), lambda b: (b, 0, 0)),
        out_shape=jax.ShapeDtypeStruct((B, 1, OUT_LEN), jnp.float32),
    )(*([x] * len(_TC_PANELS)))
    return out.reshape(B, OUT_LEN)
